# Initial kernel scaffold; baseline (speedup 1.0000x reference)
import jax
import jax.numpy as jnp
from jax.experimental import pallas as pl


def kernel(values_l4, values_l6, segment_ids):
    raise NotImplementedError("write your pallas kernel here")


# speedup vs baseline: 19.6057x; 19.6057x over previous
import functools

import jax
import jax.numpy as jnp
from jax import lax
from jax.experimental import pallas as pl
from jax.experimental.pallas import tpu as pltpu
from jax.experimental.pallas import tpu_sc as plsc

N = 160000
S = 10000
L = 16
R4 = 9
R6 = 13
R = R4 + R6

NW = 32
APW = N // NW
C = 40
NCHUNK = APW // C


def _sc_body(l4_hbm, l6_hbm, ids_hbm, qrows_hbm, partials_hbm, pids_hbm,
             ids_v, buf4, buf6, acc, qstage, zbuf, idstage,
             s4a, s4b, s6a, s6b):
    w = lax.axis_index("s") * 2 + lax.axis_index("c")
    base = w * APW

    pltpu.sync_copy(ids_hbm.at[pl.ds(base, APW)], ids_v.at[pl.ds(0, APW)])

    def id_at(idx):
        return ids_v[pl.ds(idx, L)][0]

    zv = jnp.zeros((L,), jnp.float32)
    for r in range(R):
        acc[r, :] = zv
        zbuf[r, :] = zv

    sem4 = (s4a, s4b)
    sem6 = (s6a, s6b)

    def start(chunk, b):
        off = base + chunk * C
        pltpu.async_copy(l4_hbm.at[pl.ds(off, C)], buf4.at[b], sem4[b])
        pltpu.async_copy(l6_hbm.at[pl.ds(off, C)], buf6.at[b], sem6[b])

    def wait(b):
        pltpu.make_async_copy(l4_hbm.at[pl.ds(0, C)], buf4.at[b], sem4[b]).wait()
        pltpu.make_async_copy(l6_hbm.at[pl.ds(0, C)], buf6.at[b], sem6[b]).wait()

    def close_run(cur_id, nclose):
        @pl.when(nclose == 0)
        def _():
            pltpu.sync_copy(acc, partials_hbm.at[2 * w])

        @pl.when(nclose > 0)
        def _():
            q4 = acc[0, :] * acc[0, :]
            for r in range(1, R4):
                q4 = q4 + acc[r, :] * acc[r, :]
            q6 = acc[R4, :] * acc[R4, :]
            for r in range(R4 + 1, R):
                q6 = q6 + acc[r, :] * acc[r, :]
            qstage[0, :] = q4
            qstage[1, :] = q6
            pltpu.sync_copy(qstage, qrows_hbm.at[cur_id])

        for r in range(R):
            acc[r, :] = jnp.zeros((L,), jnp.float32)

    def zero_gap(lo, hi):
        def zb(rr, carry):
            pltpu.sync_copy(zbuf.at[pl.ds(0, 2)], qrows_hbm.at[rr])
            return carry
        lax.fori_loop(lo + 1, hi, zb, 0)

    def process_chunk(chunk, b, carry):
        def atom(i, car):
            cid, ncl = car
            nid = id_at(chunk * C + i)
            changed = nid != cid

            @pl.when(changed)
            def _():
                close_run(cid, ncl)
                zero_gap(cid, nid)

            for r in range(R4):
                plsc.addupdate(acc.at[r], buf4[b, i, r, :])
            for r in range(R6):
                plsc.addupdate(acc.at[R4 + r], buf6[b, i, r, :])
            return (nid, ncl + changed.astype(jnp.int32))

        return lax.fori_loop(0, C, atom, carry)

    start(0, 0)
    start(1, 1)
    carry = (id_at(0), jnp.int32(0))

    def pair(j, car):
        wait(0)
        car = process_chunk(2 * j, 0, car)

        @pl.when(2 * j + 2 < NCHUNK)
        def _():
            start(2 * j + 2, 0)

        wait(1)
        car = process_chunk(2 * j + 1, 1, car)

        @pl.when(2 * j + 3 < NCHUNK)
        def _():
            start(2 * j + 3, 1)

        return car

    carry = lax.fori_loop(0, NCHUNK // 2, pair, carry)
    wait(0)
    carry = process_chunk(NCHUNK - 1, 0, carry)

    cur_id, nclose = carry
    pltpu.sync_copy(acc, partials_hbm.at[2 * w + 1])

    @pl.when(nclose == 0)
    def _():
        pltpu.sync_copy(zbuf, partials_hbm.at[2 * w])

    lane = lax.iota(jnp.int32, L)
    ids_vec = jnp.where(lane == 0, id_at(0),
                        jnp.where(lane == 1, id_at(APW - 1), 0))
    idstage[...] = ids_vec
    pltpu.sync_copy(idstage, pids_hbm.at[w])


_sc_pass = functools.partial(
    pl.kernel,
    out_type=(
        jax.ShapeDtypeStruct((S, 2, L), jnp.float32),
        jax.ShapeDtypeStruct((2 * NW, R, L), jnp.float32),
        jax.ShapeDtypeStruct((NW, L), jnp.int32),
    ),
    mesh=plsc.VectorSubcoreMesh(core_axis_name="c", subcore_axis_name="s"),
    compiler_params=pltpu.CompilerParams(use_tc_tiling_on_sc=False),
    scratch_types=[
        pltpu.VMEM((APW + L,), jnp.int32),
        pltpu.VMEM((2, C, R4, L), jnp.float32),
        pltpu.VMEM((2, C, R6, L), jnp.float32),
        pltpu.VMEM((R, L), jnp.float32),
        pltpu.VMEM((2, L), jnp.float32),
        pltpu.VMEM((R, L), jnp.float32),
        pltpu.VMEM((L,), jnp.int32),
        pltpu.SemaphoreType.DMA,
        pltpu.SemaphoreType.DMA,
        pltpu.SemaphoreType.DMA,
        pltpu.SemaphoreType.DMA,
    ],
)(_sc_body)


def _fix_body(q_ref, pmT_ref, pidc_ref, pidr_ref, f_ref, l_ref, o_ref):
    q = q_ref[...]
    base4 = jnp.sum(q[:, :L], axis=1, keepdims=True)
    base6 = jnp.sum(q[:, L:], axis=1, keepdims=True)

    pidc = pidc_ref[...]
    pidr = pidr_ref[...]
    pmT = pmT_ref[...]

    eqm = (pidc == pidr).astype(jnp.float32)
    combT = jnp.sum(pmT[:, :, None] * eqm[None, :, :], axis=1)
    q4k = jnp.sum(combT[: R4 * L, :] ** 2, axis=0, keepdims=True)
    q6k = jnp.sum(combT[R4 * L :, :] ** 2, axis=0, keepdims=True)

    iota = lax.broadcasted_iota(jnp.int32, (S, 1), 0).astype(jnp.float32)
    fv = f_ref[...]
    lv = l_ref[...]
    keep = jnp.any((iota > fv) & (iota < lv), axis=1, keepdims=True)

    match = (iota == pidr).astype(jnp.float32)
    cnt = jnp.sum(match, axis=1, keepdims=True)
    v4 = jnp.sum(match * q4k, axis=1, keepdims=True)
    v6 = jnp.sum(match * q6k, axis=1, keepdims=True)
    isb = cnt > 0.0
    safe = jnp.maximum(cnt, 1.0)
    out4 = jnp.where(isb, v4 / safe, jnp.where(keep, base4, 0.0))
    out6 = jnp.where(isb, v6 / safe, jnp.where(keep, base6, 0.0))
    o_ref[...] = jnp.concatenate([out4, out6], axis=1)


def _fixup(q2, pmT, pid_col, pid_row, fvec, lvec):
    return pl.pallas_call(
        _fix_body,
        out_shape=jax.ShapeDtypeStruct((S, 2), jnp.float32),
    )(q2, pmT, pid_col, pid_row, fvec, lvec)


def kernel(values_l4, values_l6, segment_ids):
    ids32 = segment_ids.astype(jnp.int32)
    qrows, partials, pids = _sc_pass(values_l4, values_l6, ids32)
    q2 = qrows.reshape(S, 2 * L)
    pmT = partials.reshape(2 * NW, R * L).T
    pid_col = pids[:, :2].reshape(2 * NW, 1).astype(jnp.float32)
    pid_row = pid_col.reshape(1, 2 * NW)
    fvec = pids[:, 0:1].reshape(1, NW).astype(jnp.float32)
    lvec = pids[:, 1:2].reshape(1, NW).astype(jnp.float32)
    return _fixup(q2, pmT, pid_col, pid_row, fvec, lvec)


# speedup vs baseline: 26.8041x; 1.3672x over previous
import functools

import jax
import jax.numpy as jnp
from jax import lax
from jax.experimental import pallas as pl
from jax.experimental.pallas import tpu as pltpu
from jax.experimental.pallas import tpu_sc as plsc

N = 160000
S = 10000
L = 16
R4 = 9
R6 = 13
R = R4 + R6

NW = 32
NT = N // 128
BIGW = 17
BIGT = 40
SMALLT = 38
MAXA = BIGT * 128


def _sc_body(l4_hbm, l6_hbm, ids_hbm, qrows_hbm, partials_hbm, pids_hbm,
             ids_v, buf4, buf6, acc, qstage, zbuf, idstage,
             s4a, s4b, s6a, s6b):
    w = lax.axis_index("s") * 2 + lax.axis_index("c")
    is_big = w < BIGW
    start_tile = SMALLT * w + 2 * jnp.minimum(w, BIGW)
    nch = jnp.where(is_big, BIGT, SMALLT)
    start_atom = start_tile * 128
    natoms = nch * 128

    @pl.when(is_big)
    def _():
        pltpu.sync_copy(ids_hbm.at[pl.ds(start_atom, BIGT * 128)],
                        ids_v.at[pl.ds(0, BIGT * 128)])

    @pl.when(jnp.logical_not(is_big))
    def _():
        pltpu.sync_copy(ids_hbm.at[pl.ds(start_atom, SMALLT * 128)],
                        ids_v.at[pl.ds(0, SMALLT * 128)])

    def id_at(idx):
        return ids_v[pl.ds(idx, L)][0]

    zv = jnp.zeros((L,), jnp.float32)
    for r in range(R):
        acc[r, :] = zv
        zbuf[r, :] = zv

    lane = lax.iota(jnp.int32, L)
    c_pt = lane // 8
    c_ps = lane % 8
    c_z = jnp.zeros((L,), jnp.int32)

    sem4 = (s4a, s4b)
    sem6 = (s6a, s6b)

    def start(chunk, b):
        at = start_tile + chunk
        pltpu.async_copy(l4_hbm.at[:, :, pl.ds(at, 1)], buf4.at[b], sem4[b])
        pltpu.async_copy(l6_hbm.at[:, :, pl.ds(at, 1)], buf6.at[b], sem6[b])

    def wait(b):
        pltpu.make_async_copy(l4_hbm.at[:, :, pl.ds(0, 1)], buf4.at[b],
                              sem4[b]).wait()
        pltpu.make_async_copy(l6_hbm.at[:, :, pl.ds(0, 1)], buf6.at[b],
                              sem6[b]).wait()

    def close_run(cur_id, nclose):
        @pl.when(nclose == 0)
        def _():
            pltpu.sync_copy(acc, partials_hbm.at[2 * w])

        @pl.when(nclose > 0)
        def _():
            q4 = acc[0, :] * acc[0, :]
            for r in range(1, R4):
                q4 = q4 + acc[r, :] * acc[r, :]
            q6 = acc[R4, :] * acc[R4, :]
            for r in range(R4 + 1, R):
                q6 = q6 + acc[r, :] * acc[r, :]
            qstage[0, :] = q4
            qstage[1, :] = q6
            pltpu.sync_copy(qstage, qrows_hbm.at[cur_id])

        for r in range(R):
            acc[r, :] = jnp.zeros((L,), jnp.float32)

    def zero_gap(lo, hi):
        def zb(rr, carry):
            pltpu.sync_copy(zbuf.at[pl.ds(0, 2)], qrows_hbm.at[rr])
            return carry
        lax.fori_loop(lo + 1, hi, zb, 0)

    def process_chunk(chunk, b, carry):
        def atom(i, car):
            cid, ncl = car
            nid = id_at(chunk * 128 + i)
            changed = nid != cid

            @pl.when(changed)
            def _():
                close_run(cid, ncl)
                zero_gap(cid, nid)

            splat_i = jnp.full((L,), i, jnp.int32)
            for r in range(R4):
                v = plsc.load_gather(
                    buf4.at[b],
                    [jnp.full((L,), r, jnp.int32), c_pt, c_z, c_ps, splat_i])
                plsc.addupdate(acc.at[r], v)
            for r in range(R6):
                v = plsc.load_gather(
                    buf6.at[b],
                    [jnp.full((L,), r, jnp.int32), c_pt, c_z, c_ps, splat_i])
                plsc.addupdate(acc.at[R4 + r], v)
            return (nid, ncl + changed.astype(jnp.int32))

        return lax.fori_loop(0, 128, atom, carry)

    start(0, 0)
    start(1, 1)
    carry = (id_at(0), jnp.int32(0))

    def pair(j, car):
        wait(0)
        car = process_chunk(2 * j, 0, car)

        @pl.when(2 * j + 2 < nch)
        def _():
            start(2 * j + 2, 0)

        wait(1)
        car = process_chunk(2 * j + 1, 1, car)

        @pl.when(2 * j + 3 < nch)
        def _():
            start(2 * j + 3, 1)

        return car

    carry = lax.fori_loop(0, nch // 2, pair, carry)

    cur_id, nclose = carry
    pltpu.sync_copy(acc, partials_hbm.at[2 * w + 1])

    @pl.when(nclose == 0)
    def _():
        pltpu.sync_copy(zbuf, partials_hbm.at[2 * w])

    ids_vec = jnp.where(lane == 0, id_at(0),
                        jnp.where(lane == 1, id_at(natoms - 1), 0))
    idstage[...] = ids_vec
    pltpu.sync_copy(idstage, pids_hbm.at[w])


_sc_pass = functools.partial(
    pl.kernel,
    out_type=(
        jax.ShapeDtypeStruct((S, 2, L), jnp.float32),
        jax.ShapeDtypeStruct((2 * NW, R, L), jnp.float32),
        jax.ShapeDtypeStruct((NW, L), jnp.int32),
    ),
    mesh=plsc.VectorSubcoreMesh(core_axis_name="c", subcore_axis_name="s"),
    compiler_params=pltpu.CompilerParams(use_tc_tiling_on_sc=False,
                                         needs_layout_passes=False),
    scratch_types=[
        pltpu.VMEM((MAXA + L,), jnp.int32),
        pltpu.VMEM((2, R4, 2, 1, 8, 128), jnp.float32),
        pltpu.VMEM((2, R6, 2, 1, 8, 128), jnp.float32),
        pltpu.VMEM((R, L), jnp.float32),
        pltpu.VMEM((2, L), jnp.float32),
        pltpu.VMEM((R, L), jnp.float32),
        pltpu.VMEM((L,), jnp.int32),
        pltpu.SemaphoreType.DMA,
        pltpu.SemaphoreType.DMA,
        pltpu.SemaphoreType.DMA,
        pltpu.SemaphoreType.DMA,
    ],
)(_sc_body)


def _fix_body(q_ref, pmT_ref, pidc_ref, pidr_ref, f_ref, l_ref, o_ref):
    q = q_ref[...]
    base4 = jnp.sum(q[:, :L], axis=1, keepdims=True)
    base6 = jnp.sum(q[:, L:], axis=1, keepdims=True)

    pidc = pidc_ref[...]
    pidr = pidr_ref[...]
    pmT = pmT_ref[...]

    eqm = (pidc == pidr).astype(jnp.float32)
    combT = jnp.sum(pmT[:, :, None] * eqm[None, :, :], axis=1)
    q4k = jnp.sum(combT[: R4 * L, :] ** 2, axis=0, keepdims=True)
    q6k = jnp.sum(combT[R4 * L :, :] ** 2, axis=0, keepdims=True)

    iota = lax.broadcasted_iota(jnp.int32, (S, 1), 0).astype(jnp.float32)
    fv = f_ref[...]
    lv = l_ref[...]
    keep = jnp.any((iota > fv) & (iota < lv), axis=1, keepdims=True)

    match = (iota == pidr).astype(jnp.float32)
    cnt = jnp.sum(match, axis=1, keepdims=True)
    v4 = jnp.sum(match * q4k, axis=1, keepdims=True)
    v6 = jnp.sum(match * q6k, axis=1, keepdims=True)
    isb = cnt > 0.0
    safe = jnp.maximum(cnt, 1.0)
    out4 = jnp.where(isb, v4 / safe, jnp.where(keep, base4, 0.0))
    out6 = jnp.where(isb, v6 / safe, jnp.where(keep, base6, 0.0))
    o_ref[...] = jnp.concatenate([out4, out6], axis=1)


def _fixup(q2, pmT, pid_col, pid_row, fvec, lvec):
    return pl.pallas_call(
        _fix_body,
        out_shape=jax.ShapeDtypeStruct((S, 2), jnp.float32),
    )(q2, pmT, pid_col, pid_row, fvec, lvec)


def _sc_view(v, rr):
    t = v.transpose(1, 2, 0)
    t = t.reshape(rr, 2, 8, NT, 128)
    return t.transpose(0, 1, 3, 2, 4)


def kernel(values_l4, values_l6, segment_ids):
    ids32 = segment_ids.astype(jnp.int32)
    u4 = _sc_view(values_l4, R4).reshape(R4, 2, NT, 1, 8, 128)
    u6 = _sc_view(values_l6, R6).reshape(R6, 2, NT, 1, 8, 128)
    u4 = u4.reshape(R4, 2, NT, 8, 128)
    u6 = u6.reshape(R6, 2, NT, 8, 128)
    qrows, partials, pids = _sc_pass(u4, u6, ids32)
    q2 = qrows.reshape(S, 2 * L)
    pmT = partials.reshape(2 * NW, R * L).T
    pid_col = pids[:, :2].reshape(2 * NW, 1).astype(jnp.float32)
    pid_row = pid_col.reshape(1, 2 * NW)
    fvec = pids[:, 0:1].reshape(1, NW).astype(jnp.float32)
    lvec = pids[:, 1:2].reshape(1, NW).astype(jnp.float32)
    return _fixup(q2, pmT, pid_col, pid_row, fvec, lvec)
